# baseline (speedup 1.0000x reference)
import jax
import jax.numpy as jnp
import numpy as np
from jax import lax
from jax.experimental import pallas as pl
from jax.experimental.pallas import tpu as pltpu
from jax.experimental.pallas import tpu_sc as plsc

F32 = jnp.float32
I32 = jnp.int32
_NC, _NS, _NL = 2, 16, 16
_NW = _NC * _NS
_M16 = np.int32(-65536)
_RND = np.int32(0x8000)


def _mesh():
    return plsc.VectorSubcoreMesh(
        core_axis_name="c", subcore_axis_name="s",
        num_cores=_NC, num_subcores=_NS)


def _unpk(u):
    lo = lax.bitcast_convert_type(lax.shift_left(u, 16), F32)
    hi = lax.bitcast_convert_type(jnp.bitwise_and(u, _M16), F32)
    return lo, hi


def _pk(a, b):
    ua = lax.shift_right_logical(lax.bitcast_convert_type(a, I32) + _RND, 16)
    ub = jnp.bitwise_and(lax.bitcast_convert_type(b, I32) + _RND, _M16)
    return jnp.bitwise_or(ua, ub)


def _mm_body(x_ref, w_ref, b_ref, o_ref):
    o_ref[...] = (
        jnp.dot(x_ref[...], w_ref[...], preferred_element_type=F32)
        + b_ref[...])


def _matmul_bias(x, w, b, row_block, col_block):
    m, k = x.shape
    _, n = w.shape
    grid = (m // row_block, n // col_block)
    return pl.pallas_call(
        _mm_body,
        grid=grid,
        in_specs=[
            pl.BlockSpec((row_block, k), lambda i, j: (i, 0)),
            pl.BlockSpec((k, col_block), lambda i, j: (0, j)),
            pl.BlockSpec((1, col_block), lambda i, j: (0, j)),
        ],
        out_specs=pl.BlockSpec((row_block, col_block), lambda i, j: (i, j)),
        out_shape=jax.ShapeDtypeStruct((m, n), F32),
    )(x, w, b.reshape(1, -1))


def _mmpk_body(x_ref, w_ref, b_ref, lo_ref, hi_ref):
    h = w_ref.shape[1] // 2
    q = h // 2
    y = (jnp.dot(x_ref[...].astype(jnp.bfloat16),
                 w_ref[...].astype(jnp.bfloat16),
                 preferred_element_type=F32) + b_ref[...])
    u1 = lax.shift_right_logical(
        lax.bitcast_convert_type(y[:, :h], I32) + _RND, 16)
    u2 = jnp.bitwise_and(
        lax.bitcast_convert_type(y[:, h:], I32) + _RND, _M16)
    u = jnp.bitwise_or(u1, u2)
    lo_ref[...] = u[:, :q]
    hi_ref[...] = u[:, q:]


def _matmul_pack(x, w, b, row_block, col_block):
    m, k = x.shape
    _, n = w.shape
    grid = (m // row_block, n // col_block)
    ospec = pl.BlockSpec((row_block, col_block // 4), lambda i, j: (i, j))
    return pl.pallas_call(
        _mmpk_body,
        grid=grid,
        in_specs=[
            pl.BlockSpec((row_block, k), lambda i, j: (i, 0)),
            pl.BlockSpec((k, col_block), lambda i, j: (0, j)),
            pl.BlockSpec((1, col_block), lambda i, j: (0, j)),
        ],
        out_specs=[ospec, ospec],
        out_shape=[jax.ShapeDtypeStruct((m, n // 4), I32),
                   jax.ShapeDtypeStruct((m, n // 4), I32)],
    )(x, w, b.reshape(1, -1))


_W1 = 40


def _p1_body(eCl, eCh, hA, hB, snd, rcv, eta_out, stats_out,
             ibuf, jbuf, ab0, ab1, bb0, bb1, cl0, cl1, ch0, ch1, stats,
             si0, si1, so0, so1):
    E = eCl.shape[0]
    Dp = hA.shape[1]
    Qp = Dp // 2
    wid = lax.axis_index("s") * _NC + lax.axis_index("c")
    n_edge = ibuf.shape[0]
    n_win = n_edge // _W1
    e0 = wid * n_edge
    abufs, bbufs = (ab0, ab1), (bb0, bb1)
    clbufs, chbufs = (cl0, cl1), (ch0, ch1)
    sis, sos = (si0, si1), (so0, so1)

    pltpu.sync_copy(snd.at[pl.ds(e0, n_edge)], ibuf)
    pltpu.sync_copy(rcv.at[pl.ds(e0, n_edge)], jbuf)
    for r in range(2 * _NL):
        stats[r] = jnp.zeros((_NL,), F32)

    def idx(b, t):
        return b.at[pl.ds(t * _W1, _W1)]

    def rows(t):
        return pl.ds(e0 + t * _W1, _W1)

    def hrows(t):
        return pl.ds(E + e0 + t * _W1, _W1)

    def issue_in(t, s):
        pltpu.async_copy(hA.at[idx(ibuf, t)], abufs[s], sis[s])
        pltpu.async_copy(hB.at[idx(jbuf, t)], bbufs[s], sis[s])
        pltpu.async_copy(eCl.at[rows(t)], clbufs[s], sis[s])
        pltpu.async_copy(eCh.at[rows(t)], chbufs[s], sis[s])

    def wait_in(t, s):
        pltpu.make_async_copy(hA.at[idx(ibuf, t)], abufs[s], sis[s]).wait()
        pltpu.make_async_copy(hB.at[idx(jbuf, t)], bbufs[s], sis[s]).wait()
        pltpu.make_async_copy(eCl.at[rows(t)], clbufs[s], sis[s]).wait()
        pltpu.make_async_copy(eCh.at[rows(t)], chbufs[s], sis[s]).wait()

    def wait_out(t, s):
        pltpu.make_async_copy(clbufs[s], eta_out.at[rows(t)], sos[s]).wait()
        pltpu.make_async_copy(chbufs[s], eta_out.at[hrows(t)],
                              sos[s]).wait()

    def step(t, s):
        o = 1 - s

        @pl.when(t >= 1)
        def _wo():
            wait_out(t, o)

        @pl.when(t + 1 < n_win)
        def _nx():
            issue_in(t + 1, o)

        wait_in(t, s)
        ab, bb = abufs[s], bbufs[s]
        for blk in range(Dp // _NL):
            sl = pl.ds(blk * _NL, _NL)
            cb = clbufs[s] if blk < 4 else chbufs[s]
            cs = pl.ds((blk % 4) * _NL, _NL)

            def inner(k, carry):
                s1, q1, s2, q2 = carry
                for u in range(4):
                    ei = k * 4 + u
                    a1, a2 = _unpk(ab[ei, sl])
                    b1, b2 = _unpk(bb[ei, sl])
                    c1, c2 = _unpk(cb[ei, cs])
                    e1 = a1 + b1 + c1
                    e2 = a2 + b2 + c2
                    cb[ei, cs] = _pk(e1, e2)
                    s1 = s1 + e1
                    q1 = q1 + e1 * e1
                    s2 = s2 + e2
                    q2 = q2 + e2 * e2
                return s1, q1, s2, q2

            g = 2 * blk
            s1, q1, s2, q2 = lax.fori_loop(
                0, _W1 // 4, inner,
                (stats[g], stats[_NL + g], stats[g + 1], stats[_NL + g + 1]))
            stats[g] = s1
            stats[_NL + g] = q1
            stats[g + 1] = s2
            stats[_NL + g + 1] = q2
        pltpu.async_copy(clbufs[s], eta_out.at[rows(t)], sos[s])
        pltpu.async_copy(chbufs[s], eta_out.at[hrows(t)], sos[s])

    issue_in(0, 0)

    def body(t, _):
        @pl.when(t % 2 == 0)
        def _a():
            step(t, 0)

        @pl.when(t % 2 == 1)
        def _b():
            step(t, 1)

        return 0

    lax.fori_loop(0, n_win, body, 0)
    wait_out(n_win - 1, (n_win - 1) % 2)
    pltpu.sync_copy(stats, stats_out.at[wid])


def _p1(eCl, eCh, hA, hB, snd, rcv):
    E = eCl.shape[0]
    Qp = eCl.shape[1]
    Dp = hA.shape[1]
    n_edge = E // _NW
    return pl.kernel(
        _p1_body,
        out_type=[
            jax.ShapeDtypeStruct((2 * E, Qp), I32),
            jax.ShapeDtypeStruct((_NW, 2 * _NL, _NL), F32),
        ],
        mesh=_mesh(),
        scratch_types=[
            pltpu.VMEM((n_edge,), I32),
            pltpu.VMEM((n_edge,), I32),
            pltpu.VMEM((_W1, Dp), I32), pltpu.VMEM((_W1, Dp), I32),
            pltpu.VMEM((_W1, Dp), I32), pltpu.VMEM((_W1, Dp), I32),
            pltpu.VMEM((_W1, Qp), I32), pltpu.VMEM((_W1, Qp), I32),
            pltpu.VMEM((_W1, Qp), I32), pltpu.VMEM((_W1, Qp), I32),
            pltpu.VMEM((2 * _NL, _NL), F32),
            pltpu.SemaphoreType.DMA, pltpu.SemaphoreType.DMA,
            pltpu.SemaphoreType.DMA, pltpu.SemaphoreType.DMA,
        ],
    )(eCl, eCh, hA, hB, snd, rcv)


_W2 = 40


def _zero_accum(zbuf, accum, sid, half):
    zrows = zbuf.shape[0]
    n_chunks = accum.shape[0] // zrows
    n_k = (n_chunks + _NS - 1) // _NS

    def zrow(ei, _):
        for g in range(half // _NL):
            zbuf[ei, pl.ds(g * _NL, _NL)] = jnp.zeros((_NL,), F32)
        return 0

    lax.fori_loop(0, zrows, zrow, 0)
    for k in range(n_k):
        c = sid + _NS * k

        @pl.when(c < n_chunks)
        def _z():
            pltpu.sync_copy(zbuf, accum.at[pl.ds(c * zrows, zrows)])


def _flush_accum(accum, out, cid, sid, zrows):
    n_chunks = accum.shape[0] // zrows
    n_k = (n_chunks + _NS - 1) // _NS
    for k in range(n_k):
        c = sid + _NS * k

        @pl.when(c < n_chunks)
        def _fl():
            r0 = c * zrows
            pltpu.sync_copy(accum.at[pl.ds(r0, zrows)],
                            out.at[cid, pl.ds(r0, zrows)])


def _p2_body(eta, e_in, snd, scl2, sft2, e_out, w16_out, s0_out,
             ib0, ib1, eb0, eb1, xb0, xb1, wb0, wb1, vb0, vb1, sbuf, fbuf,
             accum, si0, si1, sx0, sx1, sw0, sw1):
    half = accum.shape[1]
    E = e_in.shape[0]
    cid = lax.axis_index("c")
    sid = lax.axis_index("s")
    col0 = cid * half
    n_edge = E // _NS
    n_win = n_edge // _W2
    e0 = sid * n_edge
    ep0 = cid * E + e0
    ibufs = (ib0, ib1)
    ebufs, xbufs = (eb0, eb1), (xb0, xb1)
    wbufs, vbufs = (wb0, wb1), (vb0, vb1)
    sis, sxs, sws = (si0, si1), (sx0, sx1), (sw0, sw1)

    pltpu.sync_copy(scl2.at[pl.ds(cid * 8, 8)], sbuf)
    pltpu.sync_copy(sft2.at[pl.ds(cid * 8, 8)], fbuf)
    _zero_accum(wb0, accum, sid, half)
    plsc.subcore_barrier()

    def rows(t):
        return pl.ds(e0 + t * _W2, _W2)

    def prows(t):
        return pl.ds(ep0 + t * _W2, _W2)

    def issue_in(t, s):
        pltpu.async_copy(snd.at[rows(t)], ibufs[s], sis[s])
        pltpu.async_copy(eta.at[prows(t)], ebufs[s], sis[s])
        pltpu.async_copy(e_in.at[rows(t), pl.ds(col0, half)], xbufs[s],
                         sis[s])

    def wait_in(t, s):
        pltpu.make_async_copy(snd.at[rows(t)], ibufs[s], sis[s]).wait()
        pltpu.make_async_copy(eta.at[prows(t)], ebufs[s], sis[s]).wait()
        pltpu.make_async_copy(e_in.at[rows(t), pl.ds(col0, half)], xbufs[s],
                              sis[s]).wait()

    def wait_out(t, s):
        pltpu.make_async_copy(xbufs[s], e_out.at[rows(t), pl.ds(col0, half)],
                              sxs[s]).wait()
        pltpu.make_async_copy(vbufs[s], w16_out.at[prows(t)], sxs[s]).wait()
        pltpu.make_async_copy(wbufs[s], accum.at[ibufs[s]], sws[s]).wait()

    def step(t, s):
        o = 1 - s

        @pl.when(t >= 1)
        def _wo():
            wait_out(t, o)

        @pl.when(t + 1 < n_win)
        def _nx():
            issue_in(t + 1, o)

        wait_in(t, s)
        eb, xb, wb, vb = ebufs[s], xbufs[s], wbufs[s], vbufs[s]
        for blk in range(half // 32):
            slp = pl.ds(blk * _NL, _NL)
            g1 = pl.ds(blk * 32, _NL)
            g2 = pl.ds(blk * 32 + _NL, _NL)
            sc1 = sbuf[2 * blk]
            sh1 = fbuf[2 * blk]
            sc2 = sbuf[2 * blk + 1]
            sh2 = fbuf[2 * blk + 1]

            def inner(k, _c):
                for u in range(8):
                    ei = k * 8 + u
                    t1, t2 = _unpk(eb[ei, slp])
                    f1 = jnp.maximum(t1 * sc1 + sh1, 0.0)
                    f2 = jnp.maximum(t2 * sc2 + sh2, 0.0)
                    eo1 = xb[ei, g1] + f1
                    eo2 = xb[ei, g2] + f2
                    xb[ei, g1] = eo1
                    xb[ei, g2] = eo2
                    w1 = 1.0 / (1.0 + jnp.exp(-eo1))
                    w2 = 1.0 / (1.0 + jnp.exp(-eo2))
                    wb[ei, g1] = w1
                    wb[ei, g2] = w2
                    vb[ei, slp] = _pk(w1, w2)
                return 0

            lax.fori_loop(0, _W2 // 8, inner, 0)
        pltpu.async_copy(xb, e_out.at[rows(t), pl.ds(col0, half)], sxs[s])
        pltpu.async_copy(vb, w16_out.at[prows(t)], sxs[s])
        pltpu.async_copy(wb, accum.at[ibufs[s]], sws[s], add=True)

    issue_in(0, 0)

    def body(t, _):
        @pl.when(t % 2 == 0)
        def _a():
            step(t, 0)

        @pl.when(t % 2 == 1)
        def _b():
            step(t, 1)

        return 0

    lax.fori_loop(0, n_win, body, 0)
    wait_out(n_win - 1, (n_win - 1) % 2)
    plsc.subcore_barrier()
    _flush_accum(accum, s0_out, cid, sid, wb0.shape[0])


def _p2(eta, e_in, snd, scale, shift, n_nodes):
    E, D = e_in.shape
    half = D // 2
    halfp = half // 2
    return pl.kernel(
        _p2_body,
        out_type=[
            jax.ShapeDtypeStruct((E, D), F32),
            jax.ShapeDtypeStruct((2 * E, D // 4), I32),
            jax.ShapeDtypeStruct((_NC, n_nodes, half), F32),
        ],
        mesh=_mesh(),
        scratch_types=[
            pltpu.VMEM((_W2,), I32), pltpu.VMEM((_W2,), I32),
            pltpu.VMEM((_W2, halfp), I32), pltpu.VMEM((_W2, halfp), I32),
            pltpu.VMEM((_W2, half), F32), pltpu.VMEM((_W2, half), F32),
            pltpu.VMEM((_W2, half), F32), pltpu.VMEM((_W2, half), F32),
            pltpu.VMEM((_W2, halfp), I32), pltpu.VMEM((_W2, halfp), I32),
            pltpu.VMEM((8, _NL), F32),
            pltpu.VMEM((8, _NL), F32),
            pltpu.VMEM_SHARED((n_nodes, half), F32),
            pltpu.SemaphoreType.DMA, pltpu.SemaphoreType.DMA,
            pltpu.SemaphoreType.DMA, pltpu.SemaphoreType.DMA,
            pltpu.SemaphoreType.DMA, pltpu.SemaphoreType.DMA,
        ],
    )(eta, e_in, snd, scale.reshape(16, 16), shift.reshape(16, 16))


def _pagg_body(w16, snd, rcv, t2, s_out,
               ib0, ib1, jbuf, wb0, wb1, tb0, tb1, mb0, mb1, accum,
               si0, si1, sw0, sw1):
    half = accum.shape[1]
    E = snd.shape[0]
    cid = lax.axis_index("c")
    sid = lax.axis_index("s")
    n_nodes = accum.shape[0]
    n_edge = jbuf.shape[0]
    n_win = n_edge // _W2
    e0 = sid * n_edge
    ep0 = cid * E + e0
    ibufs = (ib0, ib1)
    wbufs, tbufs, mbufs = (wb0, wb1), (tb0, tb1), (mb0, mb1)
    sis, sws = (si0, si1), (sw0, sw1)

    pltpu.sync_copy(rcv.at[pl.ds(e0, n_edge)], jbuf)
    _zero_accum(mb0, accum, sid, half)
    plsc.subcore_barrier()

    def rows(t):
        return pl.ds(e0 + t * _W2, _W2)

    def prows(t):
        return pl.ds(ep0 + t * _W2, _W2)

    def jidx(t):
        return jbuf.at[pl.ds(t * _W2, _W2)]

    def issue_in(t, s):
        pltpu.async_copy(snd.at[rows(t)], ibufs[s], sis[s])
        pltpu.async_copy(w16.at[prows(t)], wbufs[s], sis[s])
        pltpu.async_copy(t2.at[jidx(t)], tbufs[s], sis[s])

    def wait_in(t, s):
        pltpu.make_async_copy(snd.at[rows(t)], ibufs[s], sis[s]).wait()
        pltpu.make_async_copy(w16.at[prows(t)], wbufs[s], sis[s]).wait()
        pltpu.make_async_copy(t2.at[jidx(t)], tbufs[s], sis[s]).wait()

    def wait_out(t, s):
        pltpu.make_async_copy(mbufs[s], accum.at[ibufs[s]], sws[s]).wait()

    def step(t, s):
        o = 1 - s

        @pl.when(t >= 1)
        def _wo():
            wait_out(t, o)

        @pl.when(t + 1 < n_win)
        def _nx():
            issue_in(t + 1, o)

        wait_in(t, s)
        wb, tb, mb = wbufs[s], tbufs[s], mbufs[s]
        tcol0 = cid * (half // 2)
        for blk in range(half // 32):
            slp = pl.ds(blk * _NL, _NL)
            tsl = pl.ds(tcol0 + blk * _NL, _NL)
            g1 = pl.ds(blk * 32, _NL)
            g2 = pl.ds(blk * 32 + _NL, _NL)

            def inner(k, _c):
                for u in range(8):
                    ei = k * 8 + u
                    w1, w2 = _unpk(wb[ei, slp])
                    t1, t2 = _unpk(tb[ei, tsl])
                    mb[ei, g1] = w1 * t1
                    mb[ei, g2] = w2 * t2
                return 0

            lax.fori_loop(0, _W2 // 8, inner, 0)
        pltpu.async_copy(mb, accum.at[ibufs[s]], sws[s], add=True)

    issue_in(0, 0)

    def body(t, _):
        @pl.when(t % 2 == 0)
        def _a():
            step(t, 0)

        @pl.when(t % 2 == 1)
        def _b():
            step(t, 1)

        return 0

    lax.fori_loop(0, n_win, body, 0)
    wait_out(n_win - 1, (n_win - 1) % 2)
    plsc.subcore_barrier()
    _flush_accum(accum, s_out, cid, sid, mb0.shape[0])


def _pagg(w16, snd, rcv, t2, n_nodes):
    E = snd.shape[0]
    half = 2 * w16.shape[1]
    halfp = w16.shape[1]
    n_edge = E // _NS
    return pl.kernel(
        _pagg_body,
        out_type=jax.ShapeDtypeStruct((_NC, n_nodes, half), F32),
        mesh=_mesh(),
        scratch_types=[
            pltpu.VMEM((_W2,), I32), pltpu.VMEM((_W2,), I32),
            pltpu.VMEM((n_edge,), I32),
            pltpu.VMEM((_W2, halfp), I32), pltpu.VMEM((_W2, halfp), I32),
            pltpu.VMEM((_W2, 2 * halfp), I32),
            pltpu.VMEM((_W2, 2 * halfp), I32),
            pltpu.VMEM((_W2, half), F32), pltpu.VMEM((_W2, half), F32),
            pltpu.VMEM_SHARED((n_nodes, half), F32),
            pltpu.SemaphoreType.DMA, pltpu.SemaphoreType.DMA,
            pltpu.SemaphoreType.DMA, pltpu.SemaphoreType.DMA,
        ],
    )(w16, snd, rcv, t2)


def _f1_body(hpU_ref, s1_ref, s0_ref, snorm_ref, t_ref, inv_ref, st_ref):
    inv = 1.0 / (s0_ref[...] + 1e-6)
    t = (hpU_ref[...] + s1_ref[...] * inv) * snorm_ref[...]
    t_ref[...] = t
    inv_ref[...] = inv
    st_ref[0, :, 0:256] = jnp.sum(t, axis=0, keepdims=True)
    st_ref[0, :, 256:512] = jnp.sum(t * t, axis=0, keepdims=True)


def _f1(hpU, s1, s0, snorm, row_block):
    n, d = hpU.shape
    grid = (n // row_block,)
    spec = pl.BlockSpec((row_block, d), lambda i: (i, 0))
    return pl.pallas_call(
        _f1_body,
        grid=grid,
        in_specs=[spec, spec, spec,
                  pl.BlockSpec((row_block, 1), lambda i: (i, 0))],
        out_specs=[spec, spec,
                   pl.BlockSpec((1, 1, 2 * d), lambda i: (i, 0, 0))],
        out_shape=[
            jax.ShapeDtypeStruct((n, d), F32),
            jax.ShapeDtypeStruct((n, d), F32),
            jax.ShapeDtypeStruct((grid[0], 1, 2 * d), F32),
        ],
    )(hpU, s1, s0, snorm.reshape(n, 1))


def _f2_body(t_ref, px_ref, s2_ref, inv_ref, h_ref, p_ref, scl_ref, sft_ref,
             ho_ref, po_ref):
    nf = jnp.maximum(t_ref[...] * scl_ref[...] + sft_ref[...], 0.0)
    ho_ref[...] = h_ref[...] + nf
    po_ref[...] = p_ref[...] + jnp.tanh(px_ref[...]
                                        + s2_ref[...] * inv_ref[...])


def _f2(t, px, s2, inv, h, p, scl, sft, row_block):
    n, d = t.shape
    grid = (n // row_block,)
    spec = pl.BlockSpec((row_block, d), lambda i: (i, 0))
    vspec = pl.BlockSpec((1, d), lambda i: (0, 0))
    return pl.pallas_call(
        _f2_body,
        grid=grid,
        in_specs=[spec, spec, spec, spec, spec, spec, vspec, vspec],
        out_specs=[spec, spec],
        out_shape=[
            jax.ShapeDtypeStruct((n, d), F32),
            jax.ShapeDtypeStruct((n, d), F32),
        ],
    )(t, px, s2, inv, h, p, scl.reshape(1, d), sft.reshape(1, d))


def kernel(h, p, e, senders, receivers, snorm_n, WA, bA, WB, bB, WC, bC,
           WU, bU, WV, bV, WX, bX, WY, bY, gamma_e, beta_e, gamma_n, beta_n):
    N, D = h.shape
    E = e.shape[0]
    half = D // 2

    def pack_cols(n_tab):
        ks = np.arange(n_tab * D // 2)
        pc = ks // (D // 4)
        kp = ks % (D // 4)
        tb = pc % n_tab
        hh = pc // n_tab
        c1 = tb * D + hh * (D // 2) + (kp // 16) * 32 + kp % 16
        return c1, c1 + 16

    bC_all = bA + bB + bC
    c1, c2 = pack_cols(1)
    eCl, eCh = _matmul_pack(
        e, jnp.concatenate([WC[:, c1], WC[:, c2]], axis=1),
        jnp.concatenate([bC_all[c1], bC_all[c2]]), 2000, D)

    X = jnp.concatenate([h, p], axis=1)
    zD = jnp.zeros((D, D), F32)
    zb = jnp.zeros((D,), F32)
    Wfull = jnp.concatenate([
        jnp.concatenate([WA, WB, WV[:D], zD], axis=1),
        jnp.concatenate([zD, zD, WV[D:], WY], axis=1),
    ], axis=0)
    bfull = jnp.concatenate([zb, zb, bV, bY])
    c1, c2 = pack_cols(4)
    XWl, XWh = _matmul_pack(
        X, jnp.concatenate([Wfull[:, c1], Wfull[:, c2]], axis=1),
        jnp.concatenate([bfull[c1], bfull[c2]]), 2000, 4 * D)
    q = half // 2
    hAi = jnp.concatenate([XWl[:, 0:q], XWh[:, 0:q]], axis=1)
    hBi = jnp.concatenate([XWl[:, q:2 * q], XWh[:, q:2 * q]], axis=1)
    hpV2 = jnp.concatenate([XWl[:, 2 * q:3 * q], XWh[:, 2 * q:3 * q]],
                           axis=1)
    pY2 = jnp.concatenate([XWl[:, 3 * q:4 * q], XWh[:, 3 * q:4 * q]],
                          axis=1)

    Wn = jnp.concatenate([
        jnp.concatenate([WU[:D], zD], axis=1),
        jnp.concatenate([WU[D:], WX], axis=1),
    ], axis=0)
    bn = jnp.concatenate([bU, bX])
    XWn = _matmul_bias(X, Wn, bn, 2000, 2 * D)
    hpU = XWn[:, 0:D]
    pX = XWn[:, D:2 * D]

    etai, stats = _p1(eCl, eCh, hAi, hBi, senders, receivers)
    part = stats.reshape(_NW, 2, D)
    s_sum = jnp.sum(part[:, 0], axis=0)
    s_sq = jnp.sum(part[:, 1], axis=0)
    mean_e = s_sum / E
    var_e = s_sq / E - mean_e * mean_e
    scale_e = gamma_e * lax.rsqrt(var_e + 1e-5)
    shift_e = beta_e - mean_e * scale_e

    e_out, w16, s0c = _p2(etai, e, senders, scale_e, shift_e, N)
    s0 = jnp.concatenate([s0c[0], s0c[1]], axis=1)

    s1c = _pagg(w16, senders, receivers, hpV2, N)
    s2c = _pagg(w16, senders, receivers, pY2, N)
    s1 = jnp.concatenate([s1c[0], s1c[1]], axis=1)
    s2 = jnp.concatenate([s2c[0], s2c[1]], axis=1)

    t, inv, nst = _f1(hpU, s1, s0, snorm_n, 2000)
    nst = nst.reshape(-1, 2 * D)
    n_sum = jnp.sum(nst[:, :D], axis=0)
    n_sq = jnp.sum(nst[:, D:], axis=0)
    mean_n = n_sum / N
    var_n = n_sq / N - mean_n * mean_n
    scale_n = gamma_n * lax.rsqrt(var_n + 1e-5)
    shift_n = beta_n - mean_n * scale_n
    h_out, p_out = _f2(t, pX, s2, inv, h, p, scale_n, shift_n, 2000)

    return (h_out, p_out, e_out)


# speedup vs baseline: 1.0645x; 1.0645x over previous
import jax
import jax.numpy as jnp
import numpy as np
from jax import lax
from jax.experimental import pallas as pl
from jax.experimental.pallas import tpu as pltpu
from jax.experimental.pallas import tpu_sc as plsc

F32 = jnp.float32
I32 = jnp.int32
_NC, _NS, _NL = 2, 16, 16
_NW = _NC * _NS
_M16 = np.int32(-65536)
_RND = np.int32(0x8000)


def _mesh():
    return plsc.VectorSubcoreMesh(
        core_axis_name="c", subcore_axis_name="s",
        num_cores=_NC, num_subcores=_NS)


def _unpk(u):
    lo = lax.bitcast_convert_type(lax.shift_left(u, 16), F32)
    hi = lax.bitcast_convert_type(jnp.bitwise_and(u, _M16), F32)
    return lo, hi


def _pk(a, b):
    ua = lax.shift_right_logical(lax.bitcast_convert_type(a, I32) + _RND, 16)
    ub = jnp.bitwise_and(lax.bitcast_convert_type(b, I32) + _RND, _M16)
    return jnp.bitwise_or(ua, ub)


def _mm_body(x_ref, w_ref, b_ref, o_ref):
    o_ref[...] = (
        jnp.dot(x_ref[...], w_ref[...], preferred_element_type=F32)
        + b_ref[...])


def _matmul_bias(x, w, b, row_block, col_block):
    m, k = x.shape
    _, n = w.shape
    grid = (m // row_block, n // col_block)
    return pl.pallas_call(
        _mm_body,
        grid=grid,
        in_specs=[
            pl.BlockSpec((row_block, k), lambda i, j: (i, 0)),
            pl.BlockSpec((k, col_block), lambda i, j: (0, j)),
            pl.BlockSpec((1, col_block), lambda i, j: (0, j)),
        ],
        out_specs=pl.BlockSpec((row_block, col_block), lambda i, j: (i, j)),
        out_shape=jax.ShapeDtypeStruct((m, n), F32),
    )(x, w, b.reshape(1, -1))


def _mmpk_body(x_ref, w_ref, b_ref, lo_ref, hi_ref):
    h = w_ref.shape[1] // 2
    q = h // 2
    y = (jnp.dot(x_ref[...].astype(jnp.bfloat16),
                 w_ref[...].astype(jnp.bfloat16),
                 preferred_element_type=F32) + b_ref[...])
    u1 = lax.shift_right_logical(
        lax.bitcast_convert_type(y[:, :h], I32) + _RND, 16)
    u2 = jnp.bitwise_and(
        lax.bitcast_convert_type(y[:, h:], I32) + _RND, _M16)
    u = jnp.bitwise_or(u1, u2)
    lo_ref[...] = u[:, :q]
    hi_ref[...] = u[:, q:]


def _matmul_pack(x, w, b, row_block, col_block):
    m, k = x.shape
    _, n = w.shape
    grid = (m // row_block, n // col_block)
    ospec = pl.BlockSpec((row_block, col_block // 4), lambda i, j: (i, j))
    return pl.pallas_call(
        _mmpk_body,
        grid=grid,
        in_specs=[
            pl.BlockSpec((row_block, k), lambda i, j: (i, 0)),
            pl.BlockSpec((k, col_block), lambda i, j: (0, j)),
            pl.BlockSpec((1, col_block), lambda i, j: (0, j)),
        ],
        out_specs=[ospec, ospec],
        out_shape=[jax.ShapeDtypeStruct((m, n // 4), I32),
                   jax.ShapeDtypeStruct((m, n // 4), I32)],
    )(x, w, b.reshape(1, -1))


_W1 = 40


def _p1_body(eCl, eCh, hA, hB, snd, rcv, eta_out, stats_out,
             ibuf, jbuf, ab0, ab1, bb0, bb1, cl0, cl1, ch0, ch1, stats,
             si0, si1, so0, so1):
    E = eCl.shape[0]
    Dp = hA.shape[1]
    Qp = Dp // 2
    wid = lax.axis_index("s") * _NC + lax.axis_index("c")
    n_edge = ibuf.shape[0]
    n_win = n_edge // _W1
    e0 = wid * n_edge
    abufs, bbufs = (ab0, ab1), (bb0, bb1)
    clbufs, chbufs = (cl0, cl1), (ch0, ch1)
    sis, sos = (si0, si1), (so0, so1)

    pltpu.sync_copy(snd.at[pl.ds(e0, n_edge)], ibuf)
    pltpu.sync_copy(rcv.at[pl.ds(e0, n_edge)], jbuf)
    for r in range(2 * _NL):
        stats[r] = jnp.zeros((_NL,), F32)

    def idx(b, t):
        return b.at[pl.ds(t * _W1, _W1)]

    def rows(t):
        return pl.ds(e0 + t * _W1, _W1)

    def hrows(t):
        return pl.ds(E + e0 + t * _W1, _W1)

    def issue_in(t, s):
        pltpu.async_copy(hA.at[idx(ibuf, t)], abufs[s], sis[s])
        pltpu.async_copy(hB.at[idx(jbuf, t)], bbufs[s], sis[s])
        pltpu.async_copy(eCl.at[rows(t)], clbufs[s], sis[s])
        pltpu.async_copy(eCh.at[rows(t)], chbufs[s], sis[s])

    def wait_in(t, s):
        pltpu.make_async_copy(hA.at[idx(ibuf, t)], abufs[s], sis[s]).wait()
        pltpu.make_async_copy(hB.at[idx(jbuf, t)], bbufs[s], sis[s]).wait()
        pltpu.make_async_copy(eCl.at[rows(t)], clbufs[s], sis[s]).wait()
        pltpu.make_async_copy(eCh.at[rows(t)], chbufs[s], sis[s]).wait()

    def wait_out(t, s):
        pltpu.make_async_copy(clbufs[s], eta_out.at[rows(t)], sos[s]).wait()
        pltpu.make_async_copy(chbufs[s], eta_out.at[hrows(t)],
                              sos[s]).wait()

    def step(t, s):
        o = 1 - s

        @pl.when(t >= 1)
        def _wo():
            wait_out(t, o)

        @pl.when(t + 1 < n_win)
        def _nx():
            issue_in(t + 1, o)

        wait_in(t, s)
        ab, bb = abufs[s], bbufs[s]
        for blk in range(Dp // _NL):
            sl = pl.ds(blk * _NL, _NL)
            cb = clbufs[s] if blk < 4 else chbufs[s]
            cs = pl.ds((blk % 4) * _NL, _NL)

            def inner(k, carry):
                s1, q1, s2, q2 = carry
                for u in range(4):
                    ei = k * 4 + u
                    a1, a2 = _unpk(ab[ei, sl])
                    b1, b2 = _unpk(bb[ei, sl])
                    c1, c2 = _unpk(cb[ei, cs])
                    e1 = a1 + b1 + c1
                    e2 = a2 + b2 + c2
                    cb[ei, cs] = _pk(e1, e2)
                    s1 = s1 + e1
                    q1 = q1 + e1 * e1
                    s2 = s2 + e2
                    q2 = q2 + e2 * e2
                return s1, q1, s2, q2

            g = 2 * blk
            s1, q1, s2, q2 = lax.fori_loop(
                0, _W1 // 4, inner,
                (stats[g], stats[_NL + g], stats[g + 1], stats[_NL + g + 1]))
            stats[g] = s1
            stats[_NL + g] = q1
            stats[g + 1] = s2
            stats[_NL + g + 1] = q2
        pltpu.async_copy(clbufs[s], eta_out.at[rows(t)], sos[s])
        pltpu.async_copy(chbufs[s], eta_out.at[hrows(t)], sos[s])

    issue_in(0, 0)

    def body(t, _):
        @pl.when(t % 2 == 0)
        def _a():
            step(t, 0)

        @pl.when(t % 2 == 1)
        def _b():
            step(t, 1)

        return 0

    lax.fori_loop(0, n_win, body, 0)
    wait_out(n_win - 1, (n_win - 1) % 2)
    pltpu.sync_copy(stats, stats_out.at[wid])


def _p1(eCl, eCh, hA, hB, snd, rcv):
    E = eCl.shape[0]
    Qp = eCl.shape[1]
    Dp = hA.shape[1]
    n_edge = E // _NW
    return pl.kernel(
        _p1_body,
        out_type=[
            jax.ShapeDtypeStruct((2 * E, Qp), I32),
            jax.ShapeDtypeStruct((_NW, 2 * _NL, _NL), F32),
        ],
        mesh=_mesh(),
        scratch_types=[
            pltpu.VMEM((n_edge,), I32),
            pltpu.VMEM((n_edge,), I32),
            pltpu.VMEM((_W1, Dp), I32), pltpu.VMEM((_W1, Dp), I32),
            pltpu.VMEM((_W1, Dp), I32), pltpu.VMEM((_W1, Dp), I32),
            pltpu.VMEM((_W1, Qp), I32), pltpu.VMEM((_W1, Qp), I32),
            pltpu.VMEM((_W1, Qp), I32), pltpu.VMEM((_W1, Qp), I32),
            pltpu.VMEM((2 * _NL, _NL), F32),
            pltpu.SemaphoreType.DMA, pltpu.SemaphoreType.DMA,
            pltpu.SemaphoreType.DMA, pltpu.SemaphoreType.DMA,
        ],
    )(eCl, eCh, hA, hB, snd, rcv)


_W2 = 40


def _zero_accum(zbuf, accum, sid, half):
    zrows = zbuf.shape[0]
    n_chunks = accum.shape[0] // zrows
    n_k = (n_chunks + _NS - 1) // _NS

    def zrow(ei, _):
        for g in range(half // _NL):
            zbuf[ei, pl.ds(g * _NL, _NL)] = jnp.zeros((_NL,), F32)
        return 0

    lax.fori_loop(0, zrows, zrow, 0)
    for k in range(n_k):
        c = sid + _NS * k

        @pl.when(c < n_chunks)
        def _z():
            pltpu.sync_copy(zbuf, accum.at[pl.ds(c * zrows, zrows)])


def _flush_accum(accum, out, cid, sid, zrows):
    n_chunks = accum.shape[0] // zrows
    n_k = (n_chunks + _NS - 1) // _NS
    for k in range(n_k):
        c = sid + _NS * k

        @pl.when(c < n_chunks)
        def _fl():
            r0 = c * zrows
            pltpu.sync_copy(accum.at[pl.ds(r0, zrows)],
                            out.at[cid, pl.ds(r0, zrows)])


def _p2_body(eta, e_in, snd, scl2, sft2, e_out, w16_out, s0_out,
             ib0, ib1, eb0, eb1, xb0, xb1, wb0, wb1, vb0, vb1, sbuf, fbuf,
             accum, si0, si1, sx0, sx1, sw0, sw1):
    half = accum.shape[1]
    E = e_in.shape[0]
    cid = lax.axis_index("c")
    sid = lax.axis_index("s")
    col0 = cid * half
    n_edge = E // _NS
    n_win = n_edge // _W2
    e0 = sid * n_edge
    ep0 = cid * E + e0
    ibufs = (ib0, ib1)
    ebufs, xbufs = (eb0, eb1), (xb0, xb1)
    wbufs, vbufs = (wb0, wb1), (vb0, vb1)
    sis, sxs, sws = (si0, si1), (sx0, sx1), (sw0, sw1)

    pltpu.sync_copy(scl2.at[pl.ds(cid * 8, 8)], sbuf)
    pltpu.sync_copy(sft2.at[pl.ds(cid * 8, 8)], fbuf)
    _zero_accum(wb0, accum, sid, half)
    plsc.subcore_barrier()

    def rows(t):
        return pl.ds(e0 + t * _W2, _W2)

    def prows(t):
        return pl.ds(ep0 + t * _W2, _W2)

    def issue_in(t, s):
        pltpu.async_copy(snd.at[rows(t)], ibufs[s], sis[s])
        pltpu.async_copy(eta.at[prows(t)], ebufs[s], sis[s])
        pltpu.async_copy(e_in.at[rows(t), pl.ds(col0, half)], xbufs[s],
                         sis[s])

    def wait_in(t, s):
        pltpu.make_async_copy(snd.at[rows(t)], ibufs[s], sis[s]).wait()
        pltpu.make_async_copy(eta.at[prows(t)], ebufs[s], sis[s]).wait()
        pltpu.make_async_copy(e_in.at[rows(t), pl.ds(col0, half)], xbufs[s],
                              sis[s]).wait()

    def wait_out(t, s):
        pltpu.make_async_copy(xbufs[s], e_out.at[rows(t), pl.ds(col0, half)],
                              sxs[s]).wait()
        pltpu.make_async_copy(vbufs[s], w16_out.at[prows(t)], sxs[s]).wait()
        pltpu.make_async_copy(wbufs[s], accum.at[ibufs[s]], sws[s]).wait()

    def step(t, s):
        o = 1 - s

        @pl.when(t >= 1)
        def _wo():
            wait_out(t, o)

        @pl.when(t + 1 < n_win)
        def _nx():
            issue_in(t + 1, o)

        wait_in(t, s)
        eb, xb, wb, vb = ebufs[s], xbufs[s], wbufs[s], vbufs[s]
        for blk in range(half // 32):
            slp = pl.ds(blk * _NL, _NL)
            g1 = pl.ds(blk * 32, _NL)
            g2 = pl.ds(blk * 32 + _NL, _NL)
            sc1 = sbuf[2 * blk]
            sh1 = fbuf[2 * blk]
            sc2 = sbuf[2 * blk + 1]
            sh2 = fbuf[2 * blk + 1]

            def inner(k, _c):
                for u in range(4):
                    ei = k * 4 + u
                    t1, t2 = _unpk(eb[ei, slp])
                    f1 = jnp.maximum(t1 * sc1 + sh1, 0.0)
                    f2 = jnp.maximum(t2 * sc2 + sh2, 0.0)
                    eo1 = xb[ei, g1] + f1
                    eo2 = xb[ei, g2] + f2
                    xb[ei, g1] = eo1
                    xb[ei, g2] = eo2
                    w1 = 1.0 / (1.0 + jnp.exp(-eo1))
                    w2 = 1.0 / (1.0 + jnp.exp(-eo2))
                    wb[ei, g1] = w1
                    wb[ei, g2] = w2
                    vb[ei, slp] = _pk(w1, w2)
                return 0

            lax.fori_loop(0, _W2 // 4, inner, 0)
        pltpu.async_copy(xb, e_out.at[rows(t), pl.ds(col0, half)], sxs[s])
        pltpu.async_copy(vb, w16_out.at[prows(t)], sxs[s])
        pltpu.async_copy(wb, accum.at[ibufs[s]], sws[s], add=True)

    issue_in(0, 0)

    def body(t, _):
        @pl.when(t % 2 == 0)
        def _a():
            step(t, 0)

        @pl.when(t % 2 == 1)
        def _b():
            step(t, 1)

        return 0

    lax.fori_loop(0, n_win, body, 0)
    wait_out(n_win - 1, (n_win - 1) % 2)
    plsc.subcore_barrier()
    _flush_accum(accum, s0_out, cid, sid, wb0.shape[0])


def _p2(eta, e_in, snd, scale, shift, n_nodes):
    E, D = e_in.shape
    half = D // 2
    halfp = half // 2
    return pl.kernel(
        _p2_body,
        out_type=[
            jax.ShapeDtypeStruct((E, D), F32),
            jax.ShapeDtypeStruct((2 * E, D // 4), I32),
            jax.ShapeDtypeStruct((_NC, n_nodes, half), F32),
        ],
        mesh=_mesh(),
        scratch_types=[
            pltpu.VMEM((_W2,), I32), pltpu.VMEM((_W2,), I32),
            pltpu.VMEM((_W2, halfp), I32), pltpu.VMEM((_W2, halfp), I32),
            pltpu.VMEM((_W2, half), F32), pltpu.VMEM((_W2, half), F32),
            pltpu.VMEM((_W2, half), F32), pltpu.VMEM((_W2, half), F32),
            pltpu.VMEM((_W2, halfp), I32), pltpu.VMEM((_W2, halfp), I32),
            pltpu.VMEM((8, _NL), F32),
            pltpu.VMEM((8, _NL), F32),
            pltpu.VMEM_SHARED((n_nodes, half), F32),
            pltpu.SemaphoreType.DMA, pltpu.SemaphoreType.DMA,
            pltpu.SemaphoreType.DMA, pltpu.SemaphoreType.DMA,
            pltpu.SemaphoreType.DMA, pltpu.SemaphoreType.DMA,
        ],
    )(eta, e_in, snd, scale.reshape(16, 16), shift.reshape(16, 16))


def _pagg_body(w16, snd, rcv, t2, s_out,
               ib0, ib1, jbuf, wb0, wb1, tb0, tb1, mb0, mb1, accum,
               si0, si1, sw0, sw1):
    half = accum.shape[1]
    E = snd.shape[0]
    cid = lax.axis_index("c")
    sid = lax.axis_index("s")
    n_nodes = accum.shape[0]
    n_edge = jbuf.shape[0]
    n_win = n_edge // _W2
    e0 = sid * n_edge
    ep0 = cid * E + e0
    ibufs = (ib0, ib1)
    wbufs, tbufs, mbufs = (wb0, wb1), (tb0, tb1), (mb0, mb1)
    sis, sws = (si0, si1), (sw0, sw1)

    pltpu.sync_copy(rcv.at[pl.ds(e0, n_edge)], jbuf)
    _zero_accum(mb0, accum, sid, half)
    plsc.subcore_barrier()

    def rows(t):
        return pl.ds(e0 + t * _W2, _W2)

    def prows(t):
        return pl.ds(ep0 + t * _W2, _W2)

    def jidx(t):
        return jbuf.at[pl.ds(t * _W2, _W2)]

    def issue_in(t, s):
        pltpu.async_copy(snd.at[rows(t)], ibufs[s], sis[s])
        pltpu.async_copy(w16.at[prows(t)], wbufs[s], sis[s])
        pltpu.async_copy(t2.at[jidx(t)], tbufs[s], sis[s])

    def wait_in(t, s):
        pltpu.make_async_copy(snd.at[rows(t)], ibufs[s], sis[s]).wait()
        pltpu.make_async_copy(w16.at[prows(t)], wbufs[s], sis[s]).wait()
        pltpu.make_async_copy(t2.at[jidx(t)], tbufs[s], sis[s]).wait()

    def wait_out(t, s):
        pltpu.make_async_copy(mbufs[s], accum.at[ibufs[s]], sws[s]).wait()

    def step(t, s):
        o = 1 - s

        @pl.when(t >= 1)
        def _wo():
            wait_out(t, o)

        @pl.when(t + 1 < n_win)
        def _nx():
            issue_in(t + 1, o)

        wait_in(t, s)
        wb, tb, mb = wbufs[s], tbufs[s], mbufs[s]
        tcol0 = cid * (half // 2)
        for blk in range(half // 32):
            slp = pl.ds(blk * _NL, _NL)
            tsl = pl.ds(tcol0 + blk * _NL, _NL)
            g1 = pl.ds(blk * 32, _NL)
            g2 = pl.ds(blk * 32 + _NL, _NL)

            def inner(k, _c):
                for u in range(4):
                    ei = k * 4 + u
                    w1, w2 = _unpk(wb[ei, slp])
                    t1, t2 = _unpk(tb[ei, tsl])
                    mb[ei, g1] = w1 * t1
                    mb[ei, g2] = w2 * t2
                return 0

            lax.fori_loop(0, _W2 // 4, inner, 0)
        pltpu.async_copy(mb, accum.at[ibufs[s]], sws[s], add=True)

    issue_in(0, 0)

    def body(t, _):
        @pl.when(t % 2 == 0)
        def _a():
            step(t, 0)

        @pl.when(t % 2 == 1)
        def _b():
            step(t, 1)

        return 0

    lax.fori_loop(0, n_win, body, 0)
    wait_out(n_win - 1, (n_win - 1) % 2)
    plsc.subcore_barrier()
    _flush_accum(accum, s_out, cid, sid, mb0.shape[0])


def _pagg(w16, snd, rcv, t2, n_nodes):
    E = snd.shape[0]
    half = 2 * w16.shape[1]
    halfp = w16.shape[1]
    n_edge = E // _NS
    return pl.kernel(
        _pagg_body,
        out_type=jax.ShapeDtypeStruct((_NC, n_nodes, half), F32),
        mesh=_mesh(),
        scratch_types=[
            pltpu.VMEM((_W2,), I32), pltpu.VMEM((_W2,), I32),
            pltpu.VMEM((n_edge,), I32),
            pltpu.VMEM((_W2, halfp), I32), pltpu.VMEM((_W2, halfp), I32),
            pltpu.VMEM((_W2, 2 * halfp), I32),
            pltpu.VMEM((_W2, 2 * halfp), I32),
            pltpu.VMEM((_W2, half), F32), pltpu.VMEM((_W2, half), F32),
            pltpu.VMEM_SHARED((n_nodes, half), F32),
            pltpu.SemaphoreType.DMA, pltpu.SemaphoreType.DMA,
            pltpu.SemaphoreType.DMA, pltpu.SemaphoreType.DMA,
        ],
    )(w16, snd, rcv, t2)


def _f1_body(hpU_ref, s1_ref, s0_ref, snorm_ref, t_ref, inv_ref, st_ref):
    inv = 1.0 / (s0_ref[...] + 1e-6)
    t = (hpU_ref[...] + s1_ref[...] * inv) * snorm_ref[...]
    t_ref[...] = t
    inv_ref[...] = inv
    st_ref[0, :, 0:256] = jnp.sum(t, axis=0, keepdims=True)
    st_ref[0, :, 256:512] = jnp.sum(t * t, axis=0, keepdims=True)


def _f1(hpU, s1, s0, snorm, row_block):
    n, d = hpU.shape
    grid = (n // row_block,)
    spec = pl.BlockSpec((row_block, d), lambda i: (i, 0))
    return pl.pallas_call(
        _f1_body,
        grid=grid,
        in_specs=[spec, spec, spec,
                  pl.BlockSpec((row_block, 1), lambda i: (i, 0))],
        out_specs=[spec, spec,
                   pl.BlockSpec((1, 1, 2 * d), lambda i: (i, 0, 0))],
        out_shape=[
            jax.ShapeDtypeStruct((n, d), F32),
            jax.ShapeDtypeStruct((n, d), F32),
            jax.ShapeDtypeStruct((grid[0], 1, 2 * d), F32),
        ],
    )(hpU, s1, s0, snorm.reshape(n, 1))


def _f2_body(t_ref, px_ref, s2_ref, inv_ref, h_ref, p_ref, scl_ref, sft_ref,
             ho_ref, po_ref):
    nf = jnp.maximum(t_ref[...] * scl_ref[...] + sft_ref[...], 0.0)
    ho_ref[...] = h_ref[...] + nf
    po_ref[...] = p_ref[...] + jnp.tanh(px_ref[...]
                                        + s2_ref[...] * inv_ref[...])


def _f2(t, px, s2, inv, h, p, scl, sft, row_block):
    n, d = t.shape
    grid = (n // row_block,)
    spec = pl.BlockSpec((row_block, d), lambda i: (i, 0))
    vspec = pl.BlockSpec((1, d), lambda i: (0, 0))
    return pl.pallas_call(
        _f2_body,
        grid=grid,
        in_specs=[spec, spec, spec, spec, spec, spec, vspec, vspec],
        out_specs=[spec, spec],
        out_shape=[
            jax.ShapeDtypeStruct((n, d), F32),
            jax.ShapeDtypeStruct((n, d), F32),
        ],
    )(t, px, s2, inv, h, p, scl.reshape(1, d), sft.reshape(1, d))


def kernel(h, p, e, senders, receivers, snorm_n, WA, bA, WB, bB, WC, bC,
           WU, bU, WV, bV, WX, bX, WY, bY, gamma_e, beta_e, gamma_n, beta_n):
    N, D = h.shape
    E = e.shape[0]
    half = D // 2

    def pack_cols(n_tab):
        ks = np.arange(n_tab * D // 2)
        pc = ks // (D // 4)
        kp = ks % (D // 4)
        tb = pc % n_tab
        hh = pc // n_tab
        c1 = tb * D + hh * (D // 2) + (kp // 16) * 32 + kp % 16
        return c1, c1 + 16

    bC_all = bA + bB + bC
    c1, c2 = pack_cols(1)
    eCl, eCh = _matmul_pack(
        e, jnp.concatenate([WC[:, c1], WC[:, c2]], axis=1),
        jnp.concatenate([bC_all[c1], bC_all[c2]]), 2000, D)

    X = jnp.concatenate([h, p], axis=1)
    zD = jnp.zeros((D, D), F32)
    zb = jnp.zeros((D,), F32)
    Wfull = jnp.concatenate([
        jnp.concatenate([WA, WB, WV[:D], zD], axis=1),
        jnp.concatenate([zD, zD, WV[D:], WY], axis=1),
    ], axis=0)
    bfull = jnp.concatenate([zb, zb, bV, bY])
    c1, c2 = pack_cols(4)
    XWl, XWh = _matmul_pack(
        X, jnp.concatenate([Wfull[:, c1], Wfull[:, c2]], axis=1),
        jnp.concatenate([bfull[c1], bfull[c2]]), 2000, 4 * D)
    q = half // 2
    hAi = jnp.concatenate([XWl[:, 0:q], XWh[:, 0:q]], axis=1)
    hBi = jnp.concatenate([XWl[:, q:2 * q], XWh[:, q:2 * q]], axis=1)
    hpV2 = jnp.concatenate([XWl[:, 2 * q:3 * q], XWh[:, 2 * q:3 * q]],
                           axis=1)
    pY2 = jnp.concatenate([XWl[:, 3 * q:4 * q], XWh[:, 3 * q:4 * q]],
                          axis=1)

    Wn = jnp.concatenate([
        jnp.concatenate([WU[:D], zD], axis=1),
        jnp.concatenate([WU[D:], WX], axis=1),
    ], axis=0)
    bn = jnp.concatenate([bU, bX])
    XWn = _matmul_bias(X, Wn, bn, 2000, 2 * D)
    hpU = XWn[:, 0:D]
    pX = XWn[:, D:2 * D]

    etai, stats = _p1(eCl, eCh, hAi, hBi, senders, receivers)
    part = stats.reshape(_NW, 2, D)
    s_sum = jnp.sum(part[:, 0], axis=0)
    s_sq = jnp.sum(part[:, 1], axis=0)
    mean_e = s_sum / E
    var_e = s_sq / E - mean_e * mean_e
    scale_e = gamma_e * lax.rsqrt(var_e + 1e-5)
    shift_e = beta_e - mean_e * scale_e

    e_out, w16, s0c = _p2(etai, e, senders, scale_e, shift_e, N)
    s0 = jnp.concatenate([s0c[0], s0c[1]], axis=1)

    s1c = _pagg(w16, senders, receivers, hpV2, N)
    s2c = _pagg(w16, senders, receivers, pY2, N)
    s1 = jnp.concatenate([s1c[0], s1c[1]], axis=1)
    s2 = jnp.concatenate([s2c[0], s2c[1]], axis=1)

    t, inv, nst = _f1(hpU, s1, s0, snorm_n, 2000)
    nst = nst.reshape(-1, 2 * D)
    n_sum = jnp.sum(nst[:, :D], axis=0)
    n_sq = jnp.sum(nst[:, D:], axis=0)
    mean_n = n_sum / N
    var_n = n_sq / N - mean_n * mean_n
    scale_n = gamma_n * lax.rsqrt(var_n + 1e-5)
    shift_n = beta_n - mean_n * scale_n
    h_out, p_out = _f2(t, pX, s2, inv, h, p, scale_n, shift_n, 2000)

    return (h_out, p_out, e_out)
